# baseline (speedup 1.0000x reference)
import functools

import jax
import jax.numpy as jnp
from jax import lax
from jax.experimental import pallas as pl
from jax.experimental.pallas import tpu as pltpu
from jax.experimental.pallas import tpu_sc as plsc

SEQ = 2048
D = 128
B = 16
NC = 2
NS = 16
NW = NC * NS
LBLK = SEQ // NW
KV = D // 16


def _emb_body(tokens_hbm, types_hbm, pos_hbm, tok_tbl_hbm, typ_tbl_hbm,
              out_hbm, tok_idx, typ_idx, pos_v,
              buf0, buf1, fbuf0, fbuf1, obuf0, obuf1,
              ssem, gsem0, gsem1, osem0, osem1):
    wid = lax.axis_index("s") * NC + lax.axis_index("c")
    l0 = wid * LBLK
    l0a = (wid // 2) * 128
    coff = (wid % 2) * LBLK

    c1 = pltpu.async_copy(tokens_hbm.at[:, pl.ds(l0a, 128)], tok_idx, ssem)
    c2 = pltpu.async_copy(types_hbm.at[:, pl.ds(l0a, 128)], typ_idx, ssem)
    c3 = pltpu.async_copy(pos_hbm.at[pl.ds(l0, LBLK)], pos_v, ssem)
    c1.wait()
    c2.wait()

    def gathers(b, buf, fbuf, gsem):
        pltpu.async_copy(
            tok_tbl_hbm.at[tok_idx.at[b, pl.ds(coff, LBLK)]], buf, gsem
        )
        pltpu.async_copy(
            typ_tbl_hbm.at[typ_idx.at[b, pl.ds(coff, LBLK)]], fbuf, gsem
        )

    def wait_gathers(b, buf, fbuf, gsem):
        pltpu.make_async_copy(
            tok_tbl_hbm.at[tok_idx.at[b, pl.ds(coff, LBLK)]], buf, gsem
        ).wait()
        pltpu.make_async_copy(
            typ_tbl_hbm.at[typ_idx.at[b, pl.ds(coff, LBLK)]], fbuf, gsem
        ).wait()

    gathers(0, buf0, fbuf0, gsem0)
    c3.wait()

    def add_batch(buf, fbuf, obuf):
        def add_row(r, carry):
            for k in range(KV):
                s = pl.ds(k * 16, 16)
                obuf[r, s] = buf[r, s] + pos_v[r, s] + fbuf[r, s]
            return carry

        lax.fori_loop(0, LBLK, add_row, 0)

    def half(i, b, buf, fbuf, obuf, gsem, osem):
        wait_gathers(b, buf, fbuf, gsem)

        @pl.when(i > 0)
        def _():
            pltpu.make_async_copy(
                obuf, out_hbm.at[pl.ds(b * SEQ + l0, LBLK)], osem
            ).wait()

        add_batch(buf, fbuf, obuf)
        pltpu.async_copy(obuf, out_hbm.at[pl.ds(b * SEQ + l0, LBLK)], osem)

    def pair_body(i, carry):
        b0 = 2 * i
        b1 = b0 + 1
        gathers(b1, buf1, fbuf1, gsem1)
        half(i, b0, buf0, fbuf0, obuf0, gsem0, osem0)

        @pl.when(i < B // 2 - 1)
        def _():
            gathers(b0 + 2, buf0, fbuf0, gsem0)

        half(i, b1, buf1, fbuf1, obuf1, gsem1, osem1)
        return carry

    lax.fori_loop(0, B // 2, pair_body, 0)

    pltpu.make_async_copy(obuf0, out_hbm.at[pl.ds(l0, LBLK)], osem0).wait()
    pltpu.make_async_copy(obuf1, out_hbm.at[pl.ds(l0, LBLK)], osem1).wait()


def kernel(tokens, types, pos_table, tok_table, type_table):
    mesh = plsc.VectorSubcoreMesh(
        core_axis_name="c", subcore_axis_name="s", num_cores=NC, num_subcores=NS
    )
    run = functools.partial(
        pl.kernel,
        mesh=mesh,
        out_type=jax.ShapeDtypeStruct((B * SEQ, D), jnp.float32),
        scratch_types=[
            pltpu.VMEM((B, 128), jnp.int32),
            pltpu.VMEM((B, 128), jnp.int32),
            pltpu.VMEM((LBLK, D), jnp.float32),
            pltpu.VMEM((LBLK, D), jnp.float32),
            pltpu.VMEM((LBLK, D), jnp.float32),
            pltpu.VMEM((LBLK, D), jnp.float32),
            pltpu.VMEM((LBLK, D), jnp.float32),
            pltpu.VMEM((LBLK, D), jnp.float32),
            pltpu.VMEM((LBLK, D), jnp.float32),
            pltpu.SemaphoreType.DMA,
            pltpu.SemaphoreType.DMA,
            pltpu.SemaphoreType.DMA,
            pltpu.SemaphoreType.DMA,
            pltpu.SemaphoreType.DMA,
        ],
    )(_emb_body)
    out = run(tokens, types, pos_table, tok_table, type_table)
    return out.reshape(B, SEQ, D)


# speedup vs baseline: 14.3088x; 14.3088x over previous
import functools

import jax
import jax.numpy as jnp
from jax import lax
from jax.experimental import pallas as pl
from jax.experimental.pallas import tpu as pltpu
from jax.experimental.pallas import tpu_sc as plsc

SEQ = 2048
D = 128
B = 16
NC = 2
NS = 16
NW = NC * NS
LBLK = SEQ // NW
KV = D // 16


def _emb_body(tokens_hbm, types_hbm, pos_hbm, tok_tbl_hbm, typ_tbl_hbm,
              out_hbm, fused_hbm, tok_idx, typ_idx, pos_v, typ_v, fused_v,
              idx0, idx1, buf0, buf1, fbuf0, fbuf1, obuf0, obuf1,
              ssem, gsem0, gsem1, osem0, osem1):
    cid = lax.axis_index("c")
    sid = lax.axis_index("s")
    wid = sid * NC + cid
    l0 = wid * LBLK
    l0a = (wid // 2) * 128
    coff = (wid % 2) * LBLK
    sbase = wid * 2 * LBLK

    c1 = pltpu.async_copy(tokens_hbm.at[:, pl.ds(l0a, 128)], tok_idx, ssem)
    c2 = pltpu.async_copy(types_hbm.at[:, pl.ds(l0a, 128)], typ_idx, ssem)
    c3 = pltpu.async_copy(pos_hbm.at[pl.ds(l0, LBLK)], pos_v, ssem)
    c4 = pltpu.async_copy(typ_tbl_hbm, typ_v, ssem)
    c1.wait()
    c2.wait()
    c3.wait()
    c4.wait()

    def fuse_row(r, carry):
        for t in range(2):
            for k in range(KV):
                s = pl.ds(k * 16, 16)
                fused_v[t * LBLK + r, s] = pos_v[r, s] + typ_v[t, s]
        return carry

    lax.fori_loop(0, LBLK, fuse_row, 0)
    pltpu.sync_copy(fused_v, fused_hbm.at[pl.ds(sbase, 2 * LBLK)])

    iota = lax.iota(jnp.int32, 16)

    def gathers(b, idx, buf, fbuf, gsem):
        for g in range(LBLK // 16):
            tvec = typ_idx[b, pl.ds(coff + g * 16, 16)]
            idx[pl.ds(g * 16, 16)] = (sbase + g * 16) + iota + tvec * LBLK
        pltpu.async_copy(
            tok_tbl_hbm.at[tok_idx.at[b, pl.ds(coff, LBLK)]], buf, gsem
        )
        pltpu.async_copy(fused_hbm.at[idx], fbuf, gsem)

    def wait_gathers(b, idx, buf, fbuf, gsem):
        pltpu.make_async_copy(
            tok_tbl_hbm.at[tok_idx.at[b, pl.ds(coff, LBLK)]], buf, gsem
        ).wait()
        pltpu.make_async_copy(fused_hbm.at[idx], fbuf, gsem).wait()

    gathers(0, idx0, buf0, fbuf0, gsem0)

    def add_batch(buf, fbuf, obuf):
        def add_row(r, carry):
            for k in range(KV):
                s = pl.ds(k * 16, 16)
                obuf[r, s] = buf[r, s] + fbuf[r, s]
            return carry

        lax.fori_loop(0, LBLK, add_row, 0)

    def half(i, b, idx, buf, fbuf, obuf, gsem, osem):
        wait_gathers(b, idx, buf, fbuf, gsem)

        @pl.when(i > 0)
        def _():
            pltpu.make_async_copy(
                obuf, out_hbm.at[pl.ds(b * SEQ + l0, LBLK)], osem
            ).wait()

        add_batch(buf, fbuf, obuf)
        pltpu.async_copy(obuf, out_hbm.at[pl.ds(b * SEQ + l0, LBLK)], osem)

    def pair_body(i, carry):
        b0 = 2 * i
        b1 = b0 + 1
        gathers(b1, idx1, buf1, fbuf1, gsem1)
        half(i, b0, idx0, buf0, fbuf0, obuf0, gsem0, osem0)

        @pl.when(i < B // 2 - 1)
        def _():
            gathers(b0 + 2, idx0, buf0, fbuf0, gsem0)

        half(i, b1, idx1, buf1, fbuf1, obuf1, gsem1, osem1)
        return carry

    lax.fori_loop(0, B // 2, pair_body, 0)

    pltpu.make_async_copy(obuf0, out_hbm.at[pl.ds(l0, LBLK)], osem0).wait()
    pltpu.make_async_copy(obuf1, out_hbm.at[pl.ds(l0, LBLK)], osem1).wait()


def kernel(tokens, types, pos_table, tok_table, type_table):
    mesh = plsc.VectorSubcoreMesh(
        core_axis_name="c", subcore_axis_name="s", num_cores=NC, num_subcores=NS
    )
    run = functools.partial(
        pl.kernel,
        mesh=mesh,
        out_type=(
            jax.ShapeDtypeStruct((B * SEQ, D), jnp.float32),
            jax.ShapeDtypeStruct((NW * 2 * LBLK, D), jnp.float32),
        ),
        scratch_types=[
            pltpu.VMEM((B, 128), jnp.int32),
            pltpu.VMEM((B, 128), jnp.int32),
            pltpu.VMEM((LBLK, D), jnp.float32),
            pltpu.VMEM((2, D), jnp.float32),
            pltpu.VMEM((2 * LBLK, D), jnp.float32),
            pltpu.VMEM((LBLK,), jnp.int32),
            pltpu.VMEM((LBLK,), jnp.int32),
            pltpu.VMEM((LBLK, D), jnp.float32),
            pltpu.VMEM((LBLK, D), jnp.float32),
            pltpu.VMEM((LBLK, D), jnp.float32),
            pltpu.VMEM((LBLK, D), jnp.float32),
            pltpu.VMEM((LBLK, D), jnp.float32),
            pltpu.VMEM((LBLK, D), jnp.float32),
            pltpu.SemaphoreType.DMA,
            pltpu.SemaphoreType.DMA,
            pltpu.SemaphoreType.DMA,
            pltpu.SemaphoreType.DMA,
            pltpu.SemaphoreType.DMA,
        ],
    )(_emb_body)
    out, _ = run(tokens, types, pos_table, tok_table, type_table)
    return out.reshape(B, SEQ, D)
